# baseline (speedup 1.0000x reference)
import functools

import jax
import jax.numpy as jnp
from jax import lax
from jax.experimental import pallas as pl
from jax.experimental.pallas import tpu as pltpu
from jax.experimental.pallas import tpu_sc as plsc

NC = 2
NS = 16
NW = NC * NS

D = 8
CHUNK = 4000
CHUNK_G = 1000


def _mesh():
    return plsc.VectorSubcoreMesh(
        core_axis_name="c", subcore_axis_name="s", num_cores=NC)


_SC_PARAMS = pltpu.CompilerParams(use_tc_tiling_on_sc=False)


def _stripe_bounds(n_nodes):
    body = (n_nodes // NS) // 8 * 8
    tail = n_nodes - NS * body
    return body, tail


def _striped_spmem_copy(n_nodes, s, src_at, dst_at):
    body, tail = _stripe_bounds(n_nodes)
    pltpu.sync_copy(src_at(pl.ds(s * body, body)), dst_at(pl.ds(s * body, body)))
    if tail:
        @pl.when(s == NS - 1)
        def _():
            pltpu.sync_copy(src_at(pl.ds(NS * body, tail)),
                            dst_at(pl.ds(NS * body, tail)))


def _deg_body(n_nodes, n_edges, src_hbm, dst_hbm, vsrc_hbm, vdst_hbm,
              zeros_hbm, out_hbm, i0, i1, vsrc_v, vdst_v, acc, ss0, ss1):
    per_w = n_edges // NW
    n_chunks = per_w // CHUNK
    c = lax.axis_index("c")
    s = lax.axis_index("s")
    _striped_spmem_copy(n_nodes, s, lambda sl: zeros_hbm.at[sl],
                        lambda sl: acc.at[sl])
    pltpu.sync_copy(vsrc_hbm, vsrc_v)
    pltpu.sync_copy(vdst_hbm, vdst_v)
    plsc.subcore_barrier()

    base = (c * NS + s) * per_w

    pltpu.sync_copy(src_hbm.at[pl.ds(base, CHUNK)], i0)

    def outer(j, carry):
        pltpu.async_copy(vsrc_v, acc.at[i0], ss0, add=True)

        @pl.when(j > 0)
        def _():
            pltpu.make_async_copy(vdst_v, acc.at[i1], ss1).wait()
        pltpu.sync_copy(dst_hbm.at[pl.ds(base + j * CHUNK, CHUNK)], i1)

        pltpu.async_copy(vdst_v, acc.at[i1], ss1, add=True)

        @pl.when(j + 1 < n_chunks)
        def _():
            pltpu.make_async_copy(vsrc_v, acc.at[i0], ss0).wait()
            pltpu.sync_copy(src_hbm.at[pl.ds(base + (j + 1) * CHUNK, CHUNK)],
                            i0)
        return carry

    lax.fori_loop(0, n_chunks, outer, 0)
    pltpu.make_async_copy(vsrc_v, acc.at[i0], ss0).wait()
    pltpu.make_async_copy(vdst_v, acc.at[i1], ss1).wait()
    plsc.subcore_barrier()
    _striped_spmem_copy(n_nodes, s, lambda sl: acc.at[sl],
                        lambda sl: out_hbm.at[c, sl])


def _make_deg_kernel(n_nodes, n_edges):
    return pl.kernel(
        functools.partial(_deg_body, n_nodes, n_edges),
        out_type=jax.ShapeDtypeStruct((NC, n_nodes, D), jnp.float32),
        mesh=_mesh(),
        compiler_params=_SC_PARAMS,
        scratch_types=[
            pltpu.VMEM((CHUNK,), jnp.int32),
            pltpu.VMEM((CHUNK,), jnp.int32),
            pltpu.VMEM((CHUNK, D), jnp.float32),
            pltpu.VMEM((CHUNK, D), jnp.float32),
            pltpu.VMEM_SHARED((n_nodes, D), jnp.float32),
            pltpu.SemaphoreType.DMA,
            pltpu.SemaphoreType.DMA,
        ],
    )


def _agg_body(n_nodes, n_edges, table_hbm, src_hbm, dst_hbm, zeros_hbm,
              out_hbm, is0, is1, id0, id1, r0, r1, acc, tab,
              sg0, sg1, ss0, ss1):
    per_w = n_edges // NW
    n_chunks = per_w // CHUNK_G
    c = lax.axis_index("c")
    s = lax.axis_index("s")
    _striped_spmem_copy(n_nodes, s, lambda sl: zeros_hbm.at[sl],
                        lambda sl: acc.at[sl])
    _striped_spmem_copy(n_nodes, s, lambda sl: table_hbm.at[sl],
                        lambda sl: tab.at[sl])
    plsc.subcore_barrier()

    base = (c * NS + s) * per_w
    IS = (is0, is1)
    ID = (id0, id1)
    R = (r0, r1)
    SG = (sg0, sg1)
    SS = (ss0, ss1)

    pltpu.sync_copy(src_hbm.at[pl.ds(base, CHUNK_G)], is0)
    pltpu.sync_copy(dst_hbm.at[pl.ds(base, CHUNK_G)], id0)
    pltpu.async_copy(tab.at[is0], r0, sg0)

    def outer(i2, carry):
        for b in (0, 1):
            nb = 1 - b
            nxt = i2 * 2 + b + 1

            def prefetch():
                off = base + nxt * CHUNK_G
                pltpu.sync_copy(src_hbm.at[pl.ds(off, CHUNK_G)], IS[nb])
                pltpu.sync_copy(dst_hbm.at[pl.ds(off, CHUNK_G)], ID[nb])
                pltpu.async_copy(tab.at[IS[nb]], R[nb], SG[nb])

            if b == 0:
                @pl.when(i2 > 0)
                def _():
                    pltpu.make_async_copy(R[1], acc.at[ID[1]], SS[1]).wait()
                prefetch()
            else:
                @pl.when(nxt < n_chunks)
                def _():
                    pltpu.make_async_copy(R[0], acc.at[ID[0]], SS[0]).wait()
                    prefetch()

            pltpu.make_async_copy(tab.at[IS[b]], R[b], SG[b]).wait()
            pltpu.async_copy(R[b], acc.at[ID[b]], SS[b], add=True)
        return carry

    lax.fori_loop(0, n_chunks // 2, outer, 0)
    pltpu.make_async_copy(R[0], acc.at[ID[0]], SS[0]).wait()
    pltpu.make_async_copy(R[1], acc.at[ID[1]], SS[1]).wait()
    plsc.subcore_barrier()
    _striped_spmem_copy(n_nodes, s, lambda sl: acc.at[sl],
                        lambda sl: out_hbm.at[c, sl])


def _make_agg_kernel(n_nodes, n_edges):
    return pl.kernel(
        functools.partial(_agg_body, n_nodes, n_edges),
        out_type=jax.ShapeDtypeStruct((NC, n_nodes, D), jnp.float32),
        mesh=_mesh(),
        compiler_params=_SC_PARAMS,
        scratch_types=[
            pltpu.VMEM((CHUNK_G,), jnp.int32),
            pltpu.VMEM((CHUNK_G,), jnp.int32),
            pltpu.VMEM((CHUNK_G,), jnp.int32),
            pltpu.VMEM((CHUNK_G,), jnp.int32),
            pltpu.VMEM((CHUNK_G, D), jnp.float32),
            pltpu.VMEM((CHUNK_G, D), jnp.float32),
            pltpu.VMEM_SHARED((n_nodes, D), jnp.float32),
            pltpu.VMEM_SHARED((n_nodes, D), jnp.float32),
            pltpu.SemaphoreType.DMA,
            pltpu.SemaphoreType.DMA,
            pltpu.SemaphoreType.DMA,
            pltpu.SemaphoreType.DMA,
        ],
    )


def _tc_norm_body(degp_ref, x_ref, b0_ref, b1_ref, xs_ref, ns_ref, nd_ref):
    d = degp_ref[0] + degp_ref[1]
    deg_out = jnp.dot(d, b0_ref[...], preferred_element_type=jnp.float32)
    deg_in = jnp.dot(d, b1_ref[...], preferred_element_type=jnp.float32)
    ns = jnp.where(deg_out > 0, lax.rsqrt(deg_out), 0.0)
    nd = jnp.where(deg_in > 0, lax.rsqrt(deg_in), 0.0)
    xs_ref[...] = x_ref[...] * ns
    ns_ref[...] = ns
    nd_ref[...] = nd


def _tc_layer1_body(m1p_ref, ns_ref, nd_ref, w1a_ref, w1b_ref, b1a_ref,
                    b1b_ref, w2a_ref, w2b_ref, t2_ref):
    m = m1p_ref[0] + m1p_ref[1]
    ns = ns_ref[...]
    nd = nd_ref[...]
    ha = jnp.dot(m, w1a_ref[...], preferred_element_type=jnp.float32)
    hb = jnp.dot(m, w1b_ref[...], preferred_element_type=jnp.float32)
    ha = jnp.maximum(ha * nd + b1a_ref[...], 0.0) * ns
    hb = jnp.maximum(hb * nd + b1b_ref[...], 0.0) * ns
    t2_ref[...] = (
        jnp.dot(ha, w2a_ref[...], preferred_element_type=jnp.float32)
        + jnp.dot(hb, w2b_ref[...], preferred_element_type=jnp.float32))


def _tc_layer2_body(m2p_ref, nd_ref, b2_ref, out_ref):
    m = m2p_ref[0] + m2p_ref[1]
    out_ref[...] = m * nd_ref[...] + b2_ref[...]


def _whole(shape):
    return pl.BlockSpec(shape, lambda: tuple(0 for _ in shape))


def _tc_norm(degp_p, x_p, b0, b1, r):
    return pl.pallas_call(
        _tc_norm_body,
        in_specs=[_whole((NC, r, 128)), _whole((r, 128)),
                  _whole((128, 128)), _whole((128, 128))],
        out_specs=[_whole((r, 128))] * 3,
        out_shape=[jax.ShapeDtypeStruct((r, 128), jnp.float32)] * 3,
    )(degp_p, x_p, b0, b1)


def _tc_layer1(m1p_p, ns_p, nd_p, ws, r):
    return pl.pallas_call(
        _tc_layer1_body,
        in_specs=[_whole((NC, r, 128)), _whole((r, 128)), _whole((r, 128))]
        + [_whole((128, 128)), _whole((128, 128)), _whole((1, 128)),
           _whole((1, 128)), _whole((128, 128)), _whole((128, 128))],
        out_specs=_whole((r, 128)),
        out_shape=jax.ShapeDtypeStruct((r, 128), jnp.float32),
    )(m1p_p, ns_p, nd_p, *ws)


def _tc_layer2(m2p_p, nd_p, b2t, r):
    return pl.pallas_call(
        _tc_layer2_body,
        in_specs=[_whole((NC, r, 128)), _whole((r, 128)), _whole((1, 128))],
        out_specs=_whole((r, 128)),
        out_shape=jax.ShapeDtypeStruct((r, 128), jnp.float32),
    )(m2p_p, nd_p, b2t)


G = 128 // D


def kernel(x, edge_index, W1, b1, W2, b2):
    n_nodes, in_f = x.shape
    n_edges = edge_index.shape[1]
    hid_f = W1.shape[1]
    out_f = W2.shape[1]
    r = n_nodes // G
    assert n_nodes % G == 0 and n_nodes % 8 == 0
    assert n_edges % (NW * 2 * CHUNK) == 0
    assert n_edges % (NW * 2 * CHUNK_G) == 0
    assert in_f <= D and out_f <= D and hid_f <= 2 * D
    src = edge_index[0]
    dst = edge_index[1]

    f32 = jnp.float32
    x_p = jnp.pad(x.astype(f32), ((0, 0), (0, D - in_f))).reshape(r, 128)
    zeros_d = jnp.zeros((n_nodes, D), f32)
    vsrc = jnp.zeros((CHUNK, D), f32).at[:, 0].set(1.0)
    vdst = jnp.zeros((CHUNK, D), f32).at[:, 1].set(1.0)

    eye_g = jnp.eye(G, dtype=f32)
    sel0 = jnp.zeros((D, D), f32).at[0, :].set(1.0)
    sel1 = jnp.zeros((D, D), f32).at[1, :].set(1.0)
    b0 = jnp.kron(eye_g, sel0)
    b1sel = jnp.kron(eye_g, sel1)
    w1p = jnp.zeros((D, 2 * D), f32).at[:in_f, :hid_f].set(W1)
    w1a = jnp.kron(eye_g, w1p[:, :D])
    w1b = jnp.kron(eye_g, w1p[:, D:])
    b1p = jnp.zeros((2 * D,), f32).at[:hid_f].set(b1)
    b1a = jnp.tile(b1p[:D], (G,))[None, :]
    b1b = jnp.tile(b1p[D:], (G,))[None, :]
    w2p = jnp.zeros((2 * D, D), f32).at[:hid_f, :out_f].set(W2)
    w2a = jnp.kron(eye_g, w2p[:D])
    w2b = jnp.kron(eye_g, w2p[D:])
    b2t = jnp.tile(jnp.zeros((D,), f32).at[:out_f].set(b2), (G,))[None, :]

    degp = _make_deg_kernel(n_nodes, n_edges)(src, dst, vsrc, vdst, zeros_d)
    xs_p, ns_p, nd_p = _tc_norm(degp.reshape(NC, r, 128), x_p, b0, b1sel, r)

    agg = _make_agg_kernel(n_nodes, n_edges)
    m1p = agg(xs_p.reshape(n_nodes, D), src, dst, zeros_d)
    t2_p = _tc_layer1(m1p.reshape(NC, r, 128), ns_p, nd_p,
                      (w1a, w1b, b1a, b1b, w2a, w2b), r)

    m2p = agg(t2_p.reshape(n_nodes, D), src, dst, zeros_d)
    out_p = _tc_layer2(m2p.reshape(NC, r, 128), nd_p, b2t, r)
    return out_p.reshape(n_nodes, D)[:, :out_f]


# speedup vs baseline: 1.2141x; 1.2141x over previous
import functools

import jax
import jax.numpy as jnp
from jax import lax
from jax.experimental import pallas as pl
from jax.experimental.pallas import tpu as pltpu
from jax.experimental.pallas import tpu_sc as plsc

NC = 2
NS = 16
NW = NC * NS

D = 8
CHUNK = 4000
CHUNK_G = 4000


def _mesh():
    return plsc.VectorSubcoreMesh(
        core_axis_name="c", subcore_axis_name="s", num_cores=NC)


_SC_PARAMS = pltpu.CompilerParams(use_tc_tiling_on_sc=False)


def _stripe_bounds(n_nodes):
    body = (n_nodes // NS) // 8 * 8
    tail = n_nodes - NS * body
    return body, tail


def _zero_acc(n_nodes, s, zeros_hbm, acc):
    body, tail = _stripe_bounds(n_nodes)
    pltpu.sync_copy(zeros_hbm.at[pl.ds(0, body)], acc.at[pl.ds(s * body, body)])
    if tail:
        @pl.when(s == NS - 1)
        def _():
            pltpu.sync_copy(zeros_hbm.at[pl.ds(0, tail)],
                            acc.at[pl.ds(NS * body, tail)])


def _striped_spmem_copy(n_nodes, s, src_at, dst_at):
    body, tail = _stripe_bounds(n_nodes)
    pltpu.sync_copy(src_at(pl.ds(s * body, body)), dst_at(pl.ds(s * body, body)))
    if tail:
        @pl.when(s == NS - 1)
        def _():
            pltpu.sync_copy(src_at(pl.ds(NS * body, tail)),
                            dst_at(pl.ds(NS * body, tail)))


def _deg_body(n_nodes, n_edges, src_hbm, dst_hbm, vsrc_hbm, vdst_hbm,
              zeros_hbm, out_hbm, i0, i1, vsrc_v, vdst_v, acc, ss0, ss1):
    per_w = n_edges // NW
    n_chunks = per_w // CHUNK
    c = lax.axis_index("c")
    s = lax.axis_index("s")
    _zero_acc(n_nodes, s, zeros_hbm, acc)
    pltpu.sync_copy(vsrc_hbm, vsrc_v)
    pltpu.sync_copy(vdst_hbm, vdst_v)
    plsc.subcore_barrier()

    base = (c * NS + s) * per_w

    pltpu.sync_copy(src_hbm.at[pl.ds(base, CHUNK)], i0)

    def outer(j, carry):
        pltpu.async_copy(vsrc_v, acc.at[i0], ss0, add=True)

        @pl.when(j > 0)
        def _():
            pltpu.make_async_copy(vdst_v, acc.at[i1], ss1).wait()
        pltpu.sync_copy(dst_hbm.at[pl.ds(base + j * CHUNK, CHUNK)], i1)

        pltpu.async_copy(vdst_v, acc.at[i1], ss1, add=True)

        @pl.when(j + 1 < n_chunks)
        def _():
            pltpu.make_async_copy(vsrc_v, acc.at[i0], ss0).wait()
            pltpu.sync_copy(src_hbm.at[pl.ds(base + (j + 1) * CHUNK, CHUNK)],
                            i0)
        return carry

    lax.fori_loop(0, n_chunks, outer, 0)
    pltpu.make_async_copy(vsrc_v, acc.at[i0], ss0).wait()
    pltpu.make_async_copy(vdst_v, acc.at[i1], ss1).wait()
    plsc.subcore_barrier()
    _striped_spmem_copy(n_nodes, s, lambda sl: acc.at[sl],
                        lambda sl: out_hbm.at[c, sl])


def _make_deg_kernel(n_nodes, n_edges):
    return pl.kernel(
        functools.partial(_deg_body, n_nodes, n_edges),
        out_type=jax.ShapeDtypeStruct((NC, n_nodes, D), jnp.float32),
        mesh=_mesh(),
        compiler_params=_SC_PARAMS,
        scratch_types=[
            pltpu.VMEM((CHUNK,), jnp.int32),
            pltpu.VMEM((CHUNK,), jnp.int32),
            pltpu.VMEM((CHUNK, D), jnp.float32),
            pltpu.VMEM((CHUNK, D), jnp.float32),
            pltpu.VMEM_SHARED((n_nodes, D), jnp.float32),
            pltpu.SemaphoreType.DMA,
            pltpu.SemaphoreType.DMA,
        ],
    )


def _agg_body(n_nodes, n_edges, table_hbm, src_hbm, dst_hbm, zeros_hbm,
              out_hbm, is0, is1, id0, id1, r0, r1, acc,
              sg0, sg1, ss0, ss1):
    per_w = n_edges // NW
    n_chunks = per_w // CHUNK_G
    c = lax.axis_index("c")
    s = lax.axis_index("s")
    _zero_acc(n_nodes, s, zeros_hbm, acc)
    plsc.subcore_barrier()

    base = (c * NS + s) * per_w
    IS = (is0, is1)
    ID = (id0, id1)
    R = (r0, r1)
    SG = (sg0, sg1)
    SS = (ss0, ss1)

    pltpu.sync_copy(src_hbm.at[pl.ds(base, CHUNK_G)], is0)
    pltpu.sync_copy(dst_hbm.at[pl.ds(base, CHUNK_G)], id0)
    pltpu.async_copy(table_hbm.at[is0], r0, sg0)

    def outer(i2, carry):
        for b in (0, 1):
            nb = 1 - b
            nxt = i2 * 2 + b + 1

            def prefetch():
                off = base + nxt * CHUNK_G
                pltpu.sync_copy(src_hbm.at[pl.ds(off, CHUNK_G)], IS[nb])
                pltpu.sync_copy(dst_hbm.at[pl.ds(off, CHUNK_G)], ID[nb])
                pltpu.async_copy(table_hbm.at[IS[nb]], R[nb], SG[nb])

            if b == 0:
                @pl.when(i2 > 0)
                def _():
                    pltpu.make_async_copy(R[1], acc.at[ID[1]], SS[1]).wait()
                prefetch()
            else:
                @pl.when(nxt < n_chunks)
                def _():
                    pltpu.make_async_copy(R[0], acc.at[ID[0]], SS[0]).wait()
                    prefetch()

            pltpu.make_async_copy(table_hbm.at[IS[b]], R[b], SG[b]).wait()
            pltpu.async_copy(R[b], acc.at[ID[b]], SS[b], add=True)
        return carry

    lax.fori_loop(0, n_chunks // 2, outer, 0)
    pltpu.make_async_copy(R[0], acc.at[ID[0]], SS[0]).wait()
    pltpu.make_async_copy(R[1], acc.at[ID[1]], SS[1]).wait()
    plsc.subcore_barrier()
    _striped_spmem_copy(n_nodes, s, lambda sl: acc.at[sl],
                        lambda sl: out_hbm.at[c, sl])


def _make_agg_kernel(n_nodes, n_edges):
    return pl.kernel(
        functools.partial(_agg_body, n_nodes, n_edges),
        out_type=jax.ShapeDtypeStruct((NC, n_nodes, D), jnp.float32),
        mesh=_mesh(),
        compiler_params=_SC_PARAMS,
        scratch_types=[
            pltpu.VMEM((CHUNK_G,), jnp.int32),
            pltpu.VMEM((CHUNK_G,), jnp.int32),
            pltpu.VMEM((CHUNK_G,), jnp.int32),
            pltpu.VMEM((CHUNK_G,), jnp.int32),
            pltpu.VMEM((CHUNK_G, D), jnp.float32),
            pltpu.VMEM((CHUNK_G, D), jnp.float32),
            pltpu.VMEM_SHARED((n_nodes, D), jnp.float32),
            pltpu.SemaphoreType.DMA,
            pltpu.SemaphoreType.DMA,
            pltpu.SemaphoreType.DMA,
            pltpu.SemaphoreType.DMA,
        ],
    )


def _tc_norm_body(degp_ref, x_ref, b0_ref, b1_ref, xs_ref, ns_ref, nd_ref):
    d = degp_ref[0] + degp_ref[1]
    deg_out = jnp.dot(d, b0_ref[...], preferred_element_type=jnp.float32)
    deg_in = jnp.dot(d, b1_ref[...], preferred_element_type=jnp.float32)
    ns = jnp.where(deg_out > 0, lax.rsqrt(deg_out), 0.0)
    nd = jnp.where(deg_in > 0, lax.rsqrt(deg_in), 0.0)
    xs_ref[...] = x_ref[...] * ns
    ns_ref[...] = ns
    nd_ref[...] = nd


def _tc_layer1_body(m1p_ref, ns_ref, nd_ref, w1a_ref, w1b_ref, b1a_ref,
                    b1b_ref, w2a_ref, w2b_ref, t2_ref):
    m = m1p_ref[0] + m1p_ref[1]
    ns = ns_ref[...]
    nd = nd_ref[...]
    ha = jnp.dot(m, w1a_ref[...], preferred_element_type=jnp.float32)
    hb = jnp.dot(m, w1b_ref[...], preferred_element_type=jnp.float32)
    ha = jnp.maximum(ha * nd + b1a_ref[...], 0.0) * ns
    hb = jnp.maximum(hb * nd + b1b_ref[...], 0.0) * ns
    t2_ref[...] = (
        jnp.dot(ha, w2a_ref[...], preferred_element_type=jnp.float32)
        + jnp.dot(hb, w2b_ref[...], preferred_element_type=jnp.float32))


def _tc_layer2_body(m2p_ref, nd_ref, b2_ref, out_ref):
    m = m2p_ref[0] + m2p_ref[1]
    out_ref[...] = m * nd_ref[...] + b2_ref[...]


def _whole(shape):
    return pl.BlockSpec(shape, lambda: tuple(0 for _ in shape))


def _tc_norm(degp_p, x_p, b0, b1, r):
    return pl.pallas_call(
        _tc_norm_body,
        in_specs=[_whole((NC, r, 128)), _whole((r, 128)),
                  _whole((128, 128)), _whole((128, 128))],
        out_specs=[_whole((r, 128))] * 3,
        out_shape=[jax.ShapeDtypeStruct((r, 128), jnp.float32)] * 3,
    )(degp_p, x_p, b0, b1)


def _tc_layer1(m1p_p, ns_p, nd_p, ws, r):
    return pl.pallas_call(
        _tc_layer1_body,
        in_specs=[_whole((NC, r, 128)), _whole((r, 128)), _whole((r, 128))]
        + [_whole((128, 128)), _whole((128, 128)), _whole((1, 128)),
           _whole((1, 128)), _whole((128, 128)), _whole((128, 128))],
        out_specs=_whole((r, 128)),
        out_shape=jax.ShapeDtypeStruct((r, 128), jnp.float32),
    )(m1p_p, ns_p, nd_p, *ws)


def _tc_layer2(m2p_p, nd_p, b2t, r):
    return pl.pallas_call(
        _tc_layer2_body,
        in_specs=[_whole((NC, r, 128)), _whole((r, 128)), _whole((1, 128))],
        out_specs=_whole((r, 128)),
        out_shape=jax.ShapeDtypeStruct((r, 128), jnp.float32),
    )(m2p_p, nd_p, b2t)


G = 128 // D


def kernel(x, edge_index, W1, b1, W2, b2):
    n_nodes, in_f = x.shape
    n_edges = edge_index.shape[1]
    hid_f = W1.shape[1]
    out_f = W2.shape[1]
    r = n_nodes // G
    assert n_nodes % G == 0 and n_nodes % 8 == 0
    assert n_edges % (NW * 2 * CHUNK) == 0
    assert n_edges % (NW * 2 * CHUNK_G) == 0
    assert in_f <= D and out_f <= D and hid_f <= 2 * D
    src = edge_index[0]
    dst = edge_index[1]

    f32 = jnp.float32
    x_p = jnp.pad(x.astype(f32), ((0, 0), (0, D - in_f))).reshape(r, 128)
    zeros_d = jnp.zeros((_stripe_bounds(n_nodes)[0], D), f32)
    vsrc = jnp.zeros((CHUNK, D), f32).at[:, 0].set(1.0)
    vdst = jnp.zeros((CHUNK, D), f32).at[:, 1].set(1.0)

    eye_g = jnp.eye(G, dtype=f32)
    sel0 = jnp.zeros((D, D), f32).at[0, :].set(1.0)
    sel1 = jnp.zeros((D, D), f32).at[1, :].set(1.0)
    b0 = jnp.kron(eye_g, sel0)
    b1sel = jnp.kron(eye_g, sel1)
    w1p = jnp.zeros((D, 2 * D), f32).at[:in_f, :hid_f].set(W1)
    w1a = jnp.kron(eye_g, w1p[:, :D])
    w1b = jnp.kron(eye_g, w1p[:, D:])
    b1p = jnp.zeros((2 * D,), f32).at[:hid_f].set(b1)
    b1a = jnp.tile(b1p[:D], (G,))[None, :]
    b1b = jnp.tile(b1p[D:], (G,))[None, :]
    w2p = jnp.zeros((2 * D, D), f32).at[:hid_f, :out_f].set(W2)
    w2a = jnp.kron(eye_g, w2p[:D])
    w2b = jnp.kron(eye_g, w2p[D:])
    b2t = jnp.tile(jnp.zeros((D,), f32).at[:out_f].set(b2), (G,))[None, :]

    degp = _make_deg_kernel(n_nodes, n_edges)(src, dst, vsrc, vdst, zeros_d)
    xs_p, ns_p, nd_p = _tc_norm(degp.reshape(NC, r, 128), x_p, b0, b1sel, r)

    agg = _make_agg_kernel(n_nodes, n_edges)
    m1p = agg(xs_p.reshape(n_nodes, D), src, dst, zeros_d)
    t2_p = _tc_layer1(m1p.reshape(NC, r, 128), ns_p, nd_p,
                      (w1a, w1b, b1a, b1b, w2a, w2b), r)

    m2p = agg(t2_p.reshape(n_nodes, D), src, dst, zeros_d)
    out_p = _tc_layer2(m2p.reshape(NC, r, 128), nd_p, b2t, r)
    return out_p.reshape(n_nodes, D)[:, :out_f]


# speedup vs baseline: 1.2425x; 1.0234x over previous
import functools

import jax
import jax.numpy as jnp
from jax import lax
from jax.experimental import pallas as pl
from jax.experimental.pallas import tpu as pltpu
from jax.experimental.pallas import tpu_sc as plsc

NC = 2
NS = 16
NW = NC * NS

D = 8
CHUNK = 4000
CHUNK_G = 4000


def _mesh():
    return plsc.VectorSubcoreMesh(
        core_axis_name="c", subcore_axis_name="s", num_cores=NC)


_SC_PARAMS = pltpu.CompilerParams(use_tc_tiling_on_sc=False)


def _stripe_bounds(n_nodes):
    body = (n_nodes // NS) // 8 * 8
    tail = n_nodes - NS * body
    return body, tail


def _zero_acc(n_nodes, s, zeros_hbm, acc):
    body, tail = _stripe_bounds(n_nodes)
    pltpu.sync_copy(zeros_hbm.at[pl.ds(0, body)], acc.at[pl.ds(s * body, body)])
    if tail:
        @pl.when(s == NS - 1)
        def _():
            pltpu.sync_copy(zeros_hbm.at[pl.ds(0, tail)],
                            acc.at[pl.ds(NS * body, tail)])


def _striped_spmem_copy(n_nodes, s, src_at, dst_at):
    body, tail = _stripe_bounds(n_nodes)
    pltpu.sync_copy(src_at(pl.ds(s * body, body)), dst_at(pl.ds(s * body, body)))
    if tail:
        @pl.when(s == NS - 1)
        def _():
            pltpu.sync_copy(src_at(pl.ds(NS * body, tail)),
                            dst_at(pl.ds(NS * body, tail)))


def _deg_body(n_nodes, n_edges, edge_hbm, vsrc_hbm, vdst_hbm,
              zeros_hbm, out_hbm, i0, i1, vsrc_v, vdst_v, acc, ss0, ss1):
    per_w = n_edges // NW
    n_chunks = per_w // CHUNK
    c = lax.axis_index("c")
    s = lax.axis_index("s")
    _zero_acc(n_nodes, s, zeros_hbm, acc)
    pltpu.sync_copy(vsrc_hbm, vsrc_v)
    pltpu.sync_copy(vdst_hbm, vdst_v)
    plsc.subcore_barrier()

    base = (c * NS + s) * per_w

    pltpu.sync_copy(edge_hbm.at[0, pl.ds(base, CHUNK)], i0)

    def outer(j, carry):
        pltpu.async_copy(vsrc_v, acc.at[i0], ss0, add=True)

        @pl.when(j > 0)
        def _():
            pltpu.make_async_copy(vdst_v, acc.at[i1], ss1).wait()
        pltpu.sync_copy(edge_hbm.at[1, pl.ds(base + j * CHUNK, CHUNK)], i1)

        pltpu.async_copy(vdst_v, acc.at[i1], ss1, add=True)

        @pl.when(j + 1 < n_chunks)
        def _():
            pltpu.make_async_copy(vsrc_v, acc.at[i0], ss0).wait()
            pltpu.sync_copy(edge_hbm.at[0, pl.ds(base + (j + 1) * CHUNK, CHUNK)],
                            i0)
        return carry

    lax.fori_loop(0, n_chunks, outer, 0)
    pltpu.make_async_copy(vsrc_v, acc.at[i0], ss0).wait()
    pltpu.make_async_copy(vdst_v, acc.at[i1], ss1).wait()
    plsc.subcore_barrier()
    _striped_spmem_copy(n_nodes, s, lambda sl: acc.at[sl],
                        lambda sl: out_hbm.at[c, sl])


def _make_deg_kernel(n_nodes, n_edges):
    return pl.kernel(
        functools.partial(_deg_body, n_nodes, n_edges),
        out_type=jax.ShapeDtypeStruct((NC, n_nodes, D), jnp.float32),
        mesh=_mesh(),
        compiler_params=_SC_PARAMS,
        scratch_types=[
            pltpu.VMEM((CHUNK,), jnp.int32),
            pltpu.VMEM((CHUNK,), jnp.int32),
            pltpu.VMEM((CHUNK, D), jnp.float32),
            pltpu.VMEM((CHUNK, D), jnp.float32),
            pltpu.VMEM_SHARED((n_nodes, D), jnp.float32),
            pltpu.SemaphoreType.DMA,
            pltpu.SemaphoreType.DMA,
        ],
    )


def _agg_body(n_nodes, n_edges, table_hbm, edge_hbm, zeros_hbm,
              out_hbm, is0, is1, id0, id1, r0, r1, acc,
              sg0, sg1, ss0, ss1):
    per_w = n_edges // NW
    n_chunks = per_w // CHUNK_G
    c = lax.axis_index("c")
    s = lax.axis_index("s")
    _zero_acc(n_nodes, s, zeros_hbm, acc)
    plsc.subcore_barrier()

    base = (c * NS + s) * per_w
    IS = (is0, is1)
    ID = (id0, id1)
    R = (r0, r1)
    SG = (sg0, sg1)
    SS = (ss0, ss1)

    pltpu.sync_copy(edge_hbm.at[0, pl.ds(base, CHUNK_G)], is0)
    pltpu.sync_copy(edge_hbm.at[1, pl.ds(base, CHUNK_G)], id0)
    pltpu.async_copy(table_hbm.at[is0], r0, sg0)

    def outer(i2, carry):
        for b in (0, 1):
            nb = 1 - b
            nxt = i2 * 2 + b + 1

            def prefetch():
                off = base + nxt * CHUNK_G
                pltpu.sync_copy(edge_hbm.at[0, pl.ds(off, CHUNK_G)], IS[nb])
                pltpu.sync_copy(edge_hbm.at[1, pl.ds(off, CHUNK_G)], ID[nb])
                pltpu.async_copy(table_hbm.at[IS[nb]], R[nb], SG[nb])

            if b == 0:
                @pl.when(i2 > 0)
                def _():
                    pltpu.make_async_copy(R[1], acc.at[ID[1]], SS[1]).wait()
                prefetch()
            else:
                @pl.when(nxt < n_chunks)
                def _():
                    pltpu.make_async_copy(R[0], acc.at[ID[0]], SS[0]).wait()
                    prefetch()

            pltpu.make_async_copy(table_hbm.at[IS[b]], R[b], SG[b]).wait()
            pltpu.async_copy(R[b], acc.at[ID[b]], SS[b], add=True)
        return carry

    lax.fori_loop(0, n_chunks // 2, outer, 0)
    pltpu.make_async_copy(R[0], acc.at[ID[0]], SS[0]).wait()
    pltpu.make_async_copy(R[1], acc.at[ID[1]], SS[1]).wait()
    plsc.subcore_barrier()
    _striped_spmem_copy(n_nodes, s, lambda sl: acc.at[sl],
                        lambda sl: out_hbm.at[c, sl])


def _make_agg_kernel(n_nodes, n_edges):
    return pl.kernel(
        functools.partial(_agg_body, n_nodes, n_edges),
        out_type=jax.ShapeDtypeStruct((NC, n_nodes, D), jnp.float32),
        mesh=_mesh(),
        compiler_params=_SC_PARAMS,
        scratch_types=[
            pltpu.VMEM((CHUNK_G,), jnp.int32),
            pltpu.VMEM((CHUNK_G,), jnp.int32),
            pltpu.VMEM((CHUNK_G,), jnp.int32),
            pltpu.VMEM((CHUNK_G,), jnp.int32),
            pltpu.VMEM((CHUNK_G, D), jnp.float32),
            pltpu.VMEM((CHUNK_G, D), jnp.float32),
            pltpu.VMEM_SHARED((n_nodes, D), jnp.float32),
            pltpu.SemaphoreType.DMA,
            pltpu.SemaphoreType.DMA,
            pltpu.SemaphoreType.DMA,
            pltpu.SemaphoreType.DMA,
        ],
    )


def _tc_norm_body(degp_ref, x_ref, b0_ref, b1_ref, xs_ref, ns_ref, nd_ref):
    d = degp_ref[0] + degp_ref[1]
    deg_out = jnp.dot(d, b0_ref[...], preferred_element_type=jnp.float32)
    deg_in = jnp.dot(d, b1_ref[...], preferred_element_type=jnp.float32)
    ns = jnp.where(deg_out > 0, lax.rsqrt(deg_out), 0.0)
    nd = jnp.where(deg_in > 0, lax.rsqrt(deg_in), 0.0)
    xs_ref[...] = x_ref[...] * ns
    ns_ref[...] = ns
    nd_ref[...] = nd


def _tc_layer1_body(m1p_ref, ns_ref, nd_ref, w1a_ref, w1b_ref, b1a_ref,
                    b1b_ref, w2a_ref, w2b_ref, t2_ref):
    m = m1p_ref[0] + m1p_ref[1]
    ns = ns_ref[...]
    nd = nd_ref[...]
    ha = jnp.dot(m, w1a_ref[...], preferred_element_type=jnp.float32)
    hb = jnp.dot(m, w1b_ref[...], preferred_element_type=jnp.float32)
    ha = jnp.maximum(ha * nd + b1a_ref[...], 0.0) * ns
    hb = jnp.maximum(hb * nd + b1b_ref[...], 0.0) * ns
    t2_ref[...] = (
        jnp.dot(ha, w2a_ref[...], preferred_element_type=jnp.float32)
        + jnp.dot(hb, w2b_ref[...], preferred_element_type=jnp.float32))


def _tc_layer2_body(m2p_ref, nd_ref, b2_ref, out_ref):
    m = m2p_ref[0] + m2p_ref[1]
    out_ref[...] = m * nd_ref[...] + b2_ref[...]


def _whole(shape):
    return pl.BlockSpec(shape, lambda: tuple(0 for _ in shape))


def _tc_norm(degp_p, x_p, b0, b1, r):
    return pl.pallas_call(
        _tc_norm_body,
        in_specs=[_whole((NC, r, 128)), _whole((r, 128)),
                  _whole((128, 128)), _whole((128, 128))],
        out_specs=[_whole((r, 128))] * 3,
        out_shape=[jax.ShapeDtypeStruct((r, 128), jnp.float32)] * 3,
    )(degp_p, x_p, b0, b1)


def _tc_layer1(m1p_p, ns_p, nd_p, ws, r):
    return pl.pallas_call(
        _tc_layer1_body,
        in_specs=[_whole((NC, r, 128)), _whole((r, 128)), _whole((r, 128))]
        + [_whole((128, 128)), _whole((128, 128)), _whole((1, 128)),
           _whole((1, 128)), _whole((128, 128)), _whole((128, 128))],
        out_specs=_whole((r, 128)),
        out_shape=jax.ShapeDtypeStruct((r, 128), jnp.float32),
    )(m1p_p, ns_p, nd_p, *ws)


def _tc_layer2(m2p_p, nd_p, b2t, r):
    return pl.pallas_call(
        _tc_layer2_body,
        in_specs=[_whole((NC, r, 128)), _whole((r, 128)), _whole((1, 128))],
        out_specs=_whole((r, 128)),
        out_shape=jax.ShapeDtypeStruct((r, 128), jnp.float32),
    )(m2p_p, nd_p, b2t)


G = 128 // D


def kernel(x, edge_index, W1, b1, W2, b2):
    n_nodes, in_f = x.shape
    n_edges = edge_index.shape[1]
    hid_f = W1.shape[1]
    out_f = W2.shape[1]
    r = n_nodes // G
    assert n_nodes % G == 0 and n_nodes % 8 == 0
    assert n_edges % (NW * 2 * CHUNK) == 0
    assert n_edges % (NW * 2 * CHUNK_G) == 0
    assert in_f <= D and out_f <= D and hid_f <= 2 * D
    f32 = jnp.float32
    x_p = jnp.pad(x.astype(f32), ((0, 0), (0, D - in_f))).reshape(r, 128)
    zeros_d = jnp.zeros((_stripe_bounds(n_nodes)[0], D), f32)
    vsrc = jnp.zeros((CHUNK, D), f32).at[:, 0].set(1.0)
    vdst = jnp.zeros((CHUNK, D), f32).at[:, 1].set(1.0)

    eye_g = jnp.eye(G, dtype=f32)
    sel0 = jnp.zeros((D, D), f32).at[0, :].set(1.0)
    sel1 = jnp.zeros((D, D), f32).at[1, :].set(1.0)
    b0 = jnp.kron(eye_g, sel0)
    b1sel = jnp.kron(eye_g, sel1)
    w1p = jnp.zeros((D, 2 * D), f32).at[:in_f, :hid_f].set(W1)
    w1a = jnp.kron(eye_g, w1p[:, :D])
    w1b = jnp.kron(eye_g, w1p[:, D:])
    b1p = jnp.zeros((2 * D,), f32).at[:hid_f].set(b1)
    b1a = jnp.tile(b1p[:D], (G,))[None, :]
    b1b = jnp.tile(b1p[D:], (G,))[None, :]
    w2p = jnp.zeros((2 * D, D), f32).at[:hid_f, :out_f].set(W2)
    w2a = jnp.kron(eye_g, w2p[:D])
    w2b = jnp.kron(eye_g, w2p[D:])
    b2t = jnp.tile(jnp.zeros((D,), f32).at[:out_f].set(b2), (G,))[None, :]

    degp = _make_deg_kernel(n_nodes, n_edges)(edge_index, vsrc, vdst, zeros_d)
    xs_p, ns_p, nd_p = _tc_norm(degp.reshape(NC, r, 128), x_p, b0, b1sel, r)

    agg = _make_agg_kernel(n_nodes, n_edges)
    m1p = agg(xs_p.reshape(n_nodes, D), edge_index, zeros_d)
    t2_p = _tc_layer1(m1p.reshape(NC, r, 128), ns_p, nd_p,
                      (w1a, w1b, b1a, b1b, w2a, w2b), r)

    m2p = agg(t2_p.reshape(n_nodes, D), edge_index, zeros_d)
    out_p = _tc_layer2(m2p.reshape(NC, r, 128), nd_p, b2t, r)
    return out_p.reshape(n_nodes, D)[:, :out_f]


# speedup vs baseline: 1.3126x; 1.0564x over previous
import functools

import jax
import jax.numpy as jnp
from jax import lax
from jax.experimental import pallas as pl
from jax.experimental.pallas import tpu as pltpu
from jax.experimental.pallas import tpu_sc as plsc

NC = 2
NS = 16
NW = NC * NS

D = 8
CHUNK = 4000
CHUNK_G = 4000


def _mesh():
    return plsc.VectorSubcoreMesh(
        core_axis_name="c", subcore_axis_name="s", num_cores=NC)


_SC_PARAMS = pltpu.CompilerParams(use_tc_tiling_on_sc=False)


def _stripe_bounds(n_nodes):
    body = (n_nodes // NS) // 8 * 8
    tail = n_nodes - NS * body
    return body, tail


def _zero_acc(n_nodes, s, zeros_hbm, acc):
    body, tail = _stripe_bounds(n_nodes)
    pltpu.sync_copy(zeros_hbm.at[pl.ds(0, body)], acc.at[pl.ds(s * body, body)])
    if tail:
        @pl.when(s == NS - 1)
        def _():
            pltpu.sync_copy(zeros_hbm.at[pl.ds(0, tail)],
                            acc.at[pl.ds(NS * body, tail)])


def _striped_spmem_copy(n_nodes, s, src_at, dst_at):
    body, tail = _stripe_bounds(n_nodes)
    pltpu.sync_copy(src_at(pl.ds(s * body, body)), dst_at(pl.ds(s * body, body)))
    if tail:
        @pl.when(s == NS - 1)
        def _():
            pltpu.sync_copy(src_at(pl.ds(NS * body, tail)),
                            dst_at(pl.ds(NS * body, tail)))


def _deg_body(n_nodes, n_edges, edge_hbm, vsrc_hbm, vdst_hbm,
              zeros_hbm, out_hbm, i0, i1, vsrc_v, vdst_v, acc, ss0, ss1):
    per_w = n_edges // NW
    n_chunks = per_w // CHUNK
    c = lax.axis_index("c")
    s = lax.axis_index("s")
    _zero_acc(n_nodes, s, zeros_hbm, acc)
    pltpu.sync_copy(vsrc_hbm, vsrc_v)
    pltpu.sync_copy(vdst_hbm, vdst_v)
    plsc.subcore_barrier()

    base = (c * NS + s) * per_w

    pltpu.sync_copy(edge_hbm.at[0, pl.ds(base, CHUNK)], i0)

    def outer(j, carry):
        pltpu.async_copy(vsrc_v, acc.at[i0], ss0, add=True)

        @pl.when(j > 0)
        def _():
            pltpu.make_async_copy(vdst_v, acc.at[i1], ss1).wait()
        pltpu.sync_copy(edge_hbm.at[1, pl.ds(base + j * CHUNK, CHUNK)], i1)

        pltpu.async_copy(vdst_v, acc.at[i1], ss1, add=True)

        @pl.when(j + 1 < n_chunks)
        def _():
            pltpu.make_async_copy(vsrc_v, acc.at[i0], ss0).wait()
            pltpu.sync_copy(edge_hbm.at[0, pl.ds(base + (j + 1) * CHUNK, CHUNK)],
                            i0)
        return carry

    lax.fori_loop(0, n_chunks, outer, 0)
    pltpu.make_async_copy(vsrc_v, acc.at[i0], ss0).wait()
    pltpu.make_async_copy(vdst_v, acc.at[i1], ss1).wait()
    plsc.subcore_barrier()
    _striped_spmem_copy(n_nodes, s, lambda sl: acc.at[sl],
                        lambda sl: out_hbm.at[c, sl])


def _make_deg_kernel(n_nodes, n_edges):
    return pl.kernel(
        functools.partial(_deg_body, n_nodes, n_edges),
        out_type=jax.ShapeDtypeStruct((NC, n_nodes, D), jnp.float32),
        mesh=_mesh(),
        compiler_params=_SC_PARAMS,
        scratch_types=[
            pltpu.VMEM((CHUNK,), jnp.int32),
            pltpu.VMEM((CHUNK,), jnp.int32),
            pltpu.VMEM((CHUNK, D), jnp.float32),
            pltpu.VMEM((CHUNK, D), jnp.float32),
            pltpu.VMEM_SHARED((n_nodes, D), jnp.float32),
            pltpu.SemaphoreType.DMA,
            pltpu.SemaphoreType.DMA,
        ],
    )


def _agg_body(n_nodes, n_edges, table_hbm, edge_hbm, zeros_hbm,
              out_hbm, is0, is1, id0, id1, r0, r1, acc,
              sg0, sg1, ss0, ss1):
    per_w = n_edges // NW
    n_chunks = per_w // CHUNK_G
    c = lax.axis_index("c")
    s = lax.axis_index("s")
    _zero_acc(n_nodes, s, zeros_hbm, acc)
    plsc.subcore_barrier()

    base = (c * NS + s) * per_w
    IS = (is0, is1)
    ID = (id0, id1)
    R = (r0, r1)
    SG = (sg0, sg1)
    SS = (ss0, ss1)

    pltpu.sync_copy(edge_hbm.at[0, pl.ds(base, CHUNK_G)], is0)
    pltpu.sync_copy(edge_hbm.at[1, pl.ds(base, CHUNK_G)], id0)
    pltpu.async_copy(table_hbm.at[is0], r0, sg0)

    def outer(i2, carry):
        for b in (0, 1):
            nb = 1 - b
            nxt = i2 * 2 + b + 1

            def prefetch():
                off = base + nxt * CHUNK_G
                pltpu.sync_copy(edge_hbm.at[0, pl.ds(off, CHUNK_G)], IS[nb])
                pltpu.sync_copy(edge_hbm.at[1, pl.ds(off, CHUNK_G)], ID[nb])
                pltpu.async_copy(table_hbm.at[IS[nb]], R[nb], SG[nb])

            if b == 0:
                @pl.when(i2 > 0)
                def _():
                    pltpu.make_async_copy(R[1], acc.at[ID[1]], SS[1]).wait()
                prefetch()
            else:
                @pl.when(nxt < n_chunks)
                def _():
                    pltpu.make_async_copy(R[0], acc.at[ID[0]], SS[0]).wait()
                    prefetch()

            pltpu.make_async_copy(table_hbm.at[IS[b]], R[b], SG[b]).wait()
            pltpu.async_copy(R[b], acc.at[ID[b]], SS[b], add=True)
        return carry

    lax.fori_loop(0, n_chunks // 2, outer, 0)
    pltpu.make_async_copy(R[0], acc.at[ID[0]], SS[0]).wait()
    pltpu.make_async_copy(R[1], acc.at[ID[1]], SS[1]).wait()
    plsc.subcore_barrier()
    _striped_spmem_copy(n_nodes, s, lambda sl: acc.at[sl],
                        lambda sl: out_hbm.at[c, sl])


def _make_agg_kernel(n_nodes, n_edges):
    return pl.kernel(
        functools.partial(_agg_body, n_nodes, n_edges),
        out_type=jax.ShapeDtypeStruct((NC, n_nodes, D), jnp.float32),
        mesh=_mesh(),
        compiler_params=_SC_PARAMS,
        scratch_types=[
            pltpu.VMEM((CHUNK_G,), jnp.int32),
            pltpu.VMEM((CHUNK_G,), jnp.int32),
            pltpu.VMEM((CHUNK_G,), jnp.int32),
            pltpu.VMEM((CHUNK_G,), jnp.int32),
            pltpu.VMEM((CHUNK_G, D), jnp.float32),
            pltpu.VMEM((CHUNK_G, D), jnp.float32),
            pltpu.VMEM_SHARED((n_nodes, D), jnp.float32),
            pltpu.SemaphoreType.DMA,
            pltpu.SemaphoreType.DMA,
            pltpu.SemaphoreType.DMA,
            pltpu.SemaphoreType.DMA,
        ],
    )


def _tc_norm_body(degp_ref, x_ref, b0_ref, b1_ref, xs_ref, ns_ref, nd_ref):
    d = degp_ref[0] + degp_ref[1]
    deg_out = jnp.dot(d, b0_ref[...], preferred_element_type=jnp.float32)
    deg_in = jnp.dot(d, b1_ref[...], preferred_element_type=jnp.float32)
    ns = jnp.where(deg_out > 0, lax.rsqrt(deg_out), 0.0)
    nd = jnp.where(deg_in > 0, lax.rsqrt(deg_in), 0.0)
    xs_ref[...] = x_ref[...] * ns
    ns_ref[...] = ns
    nd_ref[...] = nd


def _tc_layer1_body(m1p_ref, ns_ref, nd_ref, w1a_ref, w1b_ref, b1a_ref,
                    b1b_ref, w2a_ref, w2b_ref, t2_ref):
    m = m1p_ref[0] + m1p_ref[1]
    ns = ns_ref[...]
    nd = nd_ref[...]
    ha = jnp.dot(m, w1a_ref[...], preferred_element_type=jnp.float32)
    hb = jnp.dot(m, w1b_ref[...], preferred_element_type=jnp.float32)
    ha = jnp.maximum(ha * nd + b1a_ref[...], 0.0) * ns
    hb = jnp.maximum(hb * nd + b1b_ref[...], 0.0) * ns
    t2_ref[...] = (
        jnp.dot(ha, w2a_ref[...], preferred_element_type=jnp.float32)
        + jnp.dot(hb, w2b_ref[...], preferred_element_type=jnp.float32))


def _tc_layer2_body(m2p_ref, nd_ref, b2_ref, out_ref):
    m = m2p_ref[0] + m2p_ref[1]
    out_ref[...] = m * nd_ref[...] + b2_ref[...]


def _whole(shape):
    return pl.BlockSpec(shape, lambda: tuple(0 for _ in shape))


def _tc_norm(degp_p, x_p, b0, b1, r):
    return pl.pallas_call(
        _tc_norm_body,
        in_specs=[_whole((NC, r, 128)), _whole((r, 128)),
                  _whole((128, 128)), _whole((128, 128))],
        out_specs=[_whole((r, 128))] * 3,
        out_shape=[jax.ShapeDtypeStruct((r, 128), jnp.float32)] * 3,
    )(degp_p, x_p, b0, b1)


def _tc_layer1(m1p_p, ns_p, nd_p, ws, r):
    return pl.pallas_call(
        _tc_layer1_body,
        in_specs=[_whole((NC, r, 128)), _whole((r, 128)), _whole((r, 128))]
        + [_whole((128, 128)), _whole((128, 128)), _whole((1, 128)),
           _whole((1, 128)), _whole((128, 128)), _whole((128, 128))],
        out_specs=_whole((r, 128)),
        out_shape=jax.ShapeDtypeStruct((r, 128), jnp.float32),
    )(m1p_p, ns_p, nd_p, *ws)


def _tc_layer2(m2p_p, nd_p, b2t, r):
    return pl.pallas_call(
        _tc_layer2_body,
        in_specs=[_whole((NC, r, 128)), _whole((r, 128)), _whole((1, 128))],
        out_specs=_whole((r, 128)),
        out_shape=jax.ShapeDtypeStruct((r, 128), jnp.float32),
    )(m2p_p, nd_p, b2t)


G = 128 // D


def kernel(x, edge_index, W1, b1, W2, b2):
    n_nodes, in_f = x.shape
    n_edges = edge_index.shape[1]
    hid_f = W1.shape[1]
    out_f = W2.shape[1]
    r = n_nodes // G
    assert n_nodes % G == 0 and n_nodes % 8 == 0
    assert n_edges % (NW * 2 * CHUNK) == 0
    assert n_edges % (NW * 2 * CHUNK_G) == 0
    assert in_f <= D and out_f <= D and hid_f <= 2 * D
    f32 = jnp.float32
    x_p = jnp.pad(x.astype(f32), ((0, 0), (0, D - in_f))).reshape(r, 128)
    zeros_d = jnp.zeros((_stripe_bounds(n_nodes)[0], D), f32)
    vsrc = jnp.zeros((CHUNK, D), f32).at[:, 0].set(1.0)
    vdst = jnp.zeros((CHUNK, D), f32).at[:, 1].set(1.0)

    eye_g = jnp.eye(G, dtype=f32)
    sel0 = jnp.zeros((D, D), f32).at[0, :].set(1.0)
    sel1 = jnp.zeros((D, D), f32).at[1, :].set(1.0)
    b0 = jnp.kron(eye_g, sel0)
    b1sel = jnp.kron(eye_g, sel1)
    w1p = jnp.zeros((D, 2 * D), f32).at[:in_f, :hid_f].set(W1)
    w1a = jnp.kron(eye_g, w1p[:, :D])
    w1b = jnp.kron(eye_g, w1p[:, D:])
    b1p = jnp.zeros((2 * D,), f32).at[:hid_f].set(b1)
    b1a = jnp.tile(b1p[:D], (G,))[None, :]
    b1b = jnp.tile(b1p[D:], (G,))[None, :]
    w2p = jnp.zeros((2 * D, D), f32).at[:hid_f, :out_f].set(W2)
    w2a = jnp.kron(eye_g, w2p[:D])
    w2b = jnp.kron(eye_g, w2p[D:])
    b2t = jnp.tile(jnp.zeros((D,), f32).at[:out_f].set(b2), (G,))[None, :]

    degp = _make_deg_kernel(n_nodes, n_edges)(edge_index, vsrc, vdst, zeros_d)
    xs_p, ns_p, nd_p = _tc_norm(degp.reshape(NC, r, 128), x_p, b0, b1sel, r)

    agg = _make_agg_kernel(n_nodes, n_edges)
    m1p = agg(xs_p.reshape(n_nodes, D), edge_index, zeros_d)
    t2_p = _tc_layer1(m1p.reshape(NC, r, 128), ns_p, nd_p,
                      (w1a, w1b, b1a, b1b, w2a, w2b), r)

    m2p = agg(t2_p.reshape(n_nodes, D), edge_index, zeros_d)
    out_p = _tc_layer2(m2p.reshape(NC, r, 128), nd_p, b2t, r)
    return out_p.reshape(n_nodes // G, G, D)[:, :, :out_f].reshape(
        n_nodes, out_f)
